# baseline (speedup 1.0000x reference)
import functools

import jax
import jax.numpy as jnp
from jax import lax
from jax.experimental import pallas as pl
from jax.experimental.pallas import tpu as pltpu
from jax.experimental.pallas import tpu_sc as plsc

K = 3
_SLABS = 2


_BQ = 128
_NR_PAD = 10240


def _knn_body(q_ref, rt_ref, q2_ref, r2_ref, o_ref):
    m = jnp.dot(q_ref[...], rt_ref[...], preferred_element_type=jnp.float32)
    s = (q2_ref[...][:, 0:1] + r2_ref[...]) - 2.0 * m
    idx = lax.broadcasted_iota(jnp.int32, s.shape, 1)
    cols = []
    for t in range(K):
        mn = jnp.min(s, axis=1, keepdims=True)
        i = jnp.min(jnp.where(s == mn, idx, _NR_PAD), axis=1, keepdims=True)
        cols.append(i)
        if t < K - 1:
            s = jnp.where(idx == i, jnp.float32(jnp.inf), s)
    o_ref[...] = jnp.concatenate(cols + [cols[-1]] * (8 - K), axis=1)


def _knn_top3(q8, rt8, q2, r2):
    nq = q8.shape[0]
    grid = nq // _BQ
    return pl.pallas_call(
        _knn_body,
        grid=(grid,),
        in_specs=[
            pl.BlockSpec((_BQ, 8), lambda i: (i, 0)),
            pl.BlockSpec((8, _NR_PAD), lambda i: (0, 0)),
            pl.BlockSpec((_BQ, 8), lambda i: (i, 0)),
            pl.BlockSpec((1, _NR_PAD), lambda i: (0, 0)),
        ],
        out_specs=pl.BlockSpec((_BQ, 8), lambda i: (i, 0)),
        out_shape=jax.ShapeDtypeStruct((nq, 8), jnp.int32),
    )(q8, rt8, q2, r2)


def _proj_body(x_ref, w_ref, o_ref):
    o_ref[...] = jnp.dot(x_ref[...], w_ref[...],
                         preferred_element_type=jnp.float32) * (1.0 / K)


def _proj(ref_feat, w):
    n, c_in = ref_feat.shape
    c_out = w.shape[1]
    blk = 1000
    return pl.pallas_call(
        _proj_body,
        grid=(n // blk,),
        in_specs=[
            pl.BlockSpec((blk, c_in), lambda i: (i, 0)),
            pl.BlockSpec((c_in, c_out), lambda i: (0, 0)),
        ],
        out_specs=pl.BlockSpec((blk, c_out), lambda i: (i, 0)),
        out_shape=jax.ShapeDtypeStruct((n, c_out), jnp.float32),
    )(ref_feat, w)


_NW = 32
_CH = 80


def _gather_sum(proj, idx_w, nq_pad):
    c = proj.shape[1]
    b_per_w = nq_pad // _NW
    n_chunks = b_per_w // _CH

    mesh = plsc.VectorSubcoreMesh(core_axis_name="c", subcore_axis_name="s")

    @functools.partial(
        pl.kernel,
        mesh=mesh,
        out_type=jax.ShapeDtypeStruct((nq_pad, c), jnp.float32),
        scratch_types=[
            pltpu.VMEM((3 * b_per_w,), jnp.int32),
            pltpu.VMEM((_CH, c), jnp.float32),
            pltpu.VMEM((_CH, c), jnp.float32),
            pltpu.VMEM((_CH, c), jnp.float32),
            pltpu.VMEM((_CH, c), jnp.float32),
            pltpu.VMEM((_CH, c), jnp.float32),
            pltpu.VMEM((_CH, c), jnp.float32),
            pltpu.SemaphoreType.DMA,
            pltpu.SemaphoreType.DMA,
            pltpu.SemaphoreType.DMA,
            pltpu.SemaphoreType.DMA,
        ],
    )
    def k3(proj_hbm, idx_hbm, out_hbm, idxv,
           r0a, r1a, r2a, r0b, r1b, r2b, sa, sb, oa, ob, ):
        wid = lax.axis_index("s") * 2 + lax.axis_index("c")
        base = wid * b_per_w
        pltpu.sync_copy(idx_hbm.at[wid], idxv)
        bufs = ((r0a, r1a, r2a, sa, oa), (r0b, r1b, r2b, sb, ob))

        def fire(ci):
            r0, r1, r2, sem, _ = bufs[ci % 2]
            off = ci * _CH
            return (
                pltpu.async_copy(proj_hbm.at[idxv.at[pl.ds(off, _CH)]],
                                 r0, sem),
                pltpu.async_copy(
                    proj_hbm.at[idxv.at[pl.ds(b_per_w + off, _CH)]], r1, sem),
                pltpu.async_copy(
                    proj_hbm.at[idxv.at[pl.ds(2 * b_per_w + off, _CH)]],
                    r2, sem),
            )

        gather_hs = {0: fire(0)}
        out_hs = {}
        for ci in range(n_chunks):
            r0, r1, r2, _, osem = bufs[ci % 2]
            if ci + 1 < n_chunks:
                if ci - 1 >= 0:
                    out_hs.pop(ci - 1).wait()
                gather_hs[ci + 1] = fire(ci + 1)
            for h in gather_hs.pop(ci):
                h.wait()

            def add_body(r, _):
                for j in range(c // 16):
                    sl = pl.ds(j * 16, 16)
                    r0[r, sl] = r0[r, sl] + r1[r, sl] + r2[r, sl]
                return 0

            lax.fori_loop(0, _CH, add_body, 0)
            out_hs[ci] = pltpu.async_copy(
                r0, out_hbm.at[pl.ds(base + ci * _CH, _CH)], osem)
        for ci in sorted(out_hs):
            out_hs[ci].wait()

    return k3(proj, idx_w)


def _final_body(s_ref, w_ref, g_ref, b_ref, o_ref):
    acc = jnp.dot(s_ref[...], w_ref[...], preferred_element_type=jnp.float32)
    o_ref[...] = jnp.maximum(acc + g_ref[...] + b_ref[...], 0.0)


def _final(skip, w, gsum, b):
    n, c_in = skip.shape
    c_out = w.shape[1]
    blk = next(bb for bb in (400, 200, 8) if n % bb == 0)
    return pl.pallas_call(
        _final_body,
        grid=(n // blk,),
        in_specs=[
            pl.BlockSpec((blk, c_in), lambda i: (i, 0)),
            pl.BlockSpec((c_in, c_out), lambda i: (0, 0)),
            pl.BlockSpec((blk, c_out), lambda i: (i, 0)),
            pl.BlockSpec((1, c_out), lambda i: (0, 0)),
        ],
        out_specs=pl.BlockSpec((blk, c_out), lambda i: (i, 0)),
        out_shape=jax.ShapeDtypeStruct((n, c_out), jnp.float32),
    )(skip, w, gsum, b.reshape(1, c_out))


def kernel(ref_bxyz, ref_feat, query_bxyz, query_skip_feat, W_ref, W_skip, b):
    n_ref = ref_feat.shape[0]
    n_query = query_bxyz.shape[0]

    qry = query_bxyz[:, 1:]
    rxyz = ref_bxyz[:, 1:]
    rt8 = jnp.pad(rxyz.T, ((0, 5), (0, _NR_PAD - n_ref)))
    r2 = jnp.sum(rxyz * rxyz, axis=1)
    r2 = jnp.pad(r2, (0, _NR_PAD - n_ref), constant_values=3e8)[None, :]

    proj = _proj(ref_feat, W_ref)

    n_slab = n_query // _SLABS
    outs = []
    for sl in range(_SLABS):
        qs = qry[sl * n_slab:(sl + 1) * n_slab]
        nq1 = ((n_slab + _BQ - 1) // _BQ) * _BQ
        q8 = jnp.pad(qs, ((0, nq1 - n_slab), (0, 5)))
        q2 = jnp.sum(qs * qs, axis=1, keepdims=True)
        q2 = jnp.pad(q2, ((0, nq1 - n_slab), (0, 7)), mode="edge")

        idx8 = _knn_top3(q8, rt8, q2, r2)
        idx3 = idx8[:n_slab, :K]

        nq2 = ((n_slab + _NW * _CH - 1) // (_NW * _CH)) * (_NW * _CH)
        b_per_w = nq2 // _NW
        idx_w = (jnp.pad(idx3, ((0, nq2 - n_slab), (0, 0)))
                 .reshape(_NW, b_per_w, K).transpose(0, 2, 1)
                 .reshape(_NW, K * b_per_w))
        gsum = _gather_sum(proj, idx_w, nq2)[:n_slab]

        skip = query_skip_feat[sl * n_slab:(sl + 1) * n_slab]
        outs.append(_final(skip, W_skip, gsum, b))
    return jnp.concatenate(outs, axis=0)


# speedup vs baseline: 1.0710x; 1.0710x over previous
import functools

import jax
import jax.numpy as jnp
from jax import lax
from jax.experimental import pallas as pl
from jax.experimental.pallas import tpu as pltpu
from jax.experimental.pallas import tpu_sc as plsc

K = 3
_SLABS = 2


_BQ = 256
_NR_PAD = 10240


def _knn_body(q_ref, rt_ref, q2_ref, r2_ref, o_ref):
    m = jnp.dot(q_ref[...], rt_ref[...], preferred_element_type=jnp.float32)
    s = (q2_ref[...][:, 0:1] + r2_ref[...]) - 2.0 * m
    idx = lax.broadcasted_iota(jnp.int32, s.shape, 1)
    cols = []
    for t in range(K):
        mn = jnp.min(s, axis=1, keepdims=True)
        i = jnp.min(jnp.where(s == mn, idx, _NR_PAD), axis=1, keepdims=True)
        cols.append(i)
        if t < K - 1:
            s = jnp.where(idx == i, jnp.float32(jnp.inf), s)
    o_ref[...] = jnp.concatenate(cols + [cols[-1]] * (8 - K), axis=1)


def _knn_top3(q8, rt8, q2, r2):
    nq = q8.shape[0]
    grid = nq // _BQ
    return pl.pallas_call(
        _knn_body,
        grid=(grid,),
        in_specs=[
            pl.BlockSpec((_BQ, 8), lambda i: (i, 0)),
            pl.BlockSpec((8, _NR_PAD), lambda i: (0, 0)),
            pl.BlockSpec((_BQ, 8), lambda i: (i, 0)),
            pl.BlockSpec((1, _NR_PAD), lambda i: (0, 0)),
        ],
        out_specs=pl.BlockSpec((_BQ, 8), lambda i: (i, 0)),
        out_shape=jax.ShapeDtypeStruct((nq, 8), jnp.int32),
    )(q8, rt8, q2, r2)


def _proj_body(x_ref, w_ref, o_ref):
    o_ref[...] = jnp.dot(x_ref[...], w_ref[...],
                         preferred_element_type=jnp.float32) * (1.0 / K)


def _proj(ref_feat, w):
    n, c_in = ref_feat.shape
    c_out = w.shape[1]
    blk = 1000
    return pl.pallas_call(
        _proj_body,
        grid=(n // blk,),
        in_specs=[
            pl.BlockSpec((blk, c_in), lambda i: (i, 0)),
            pl.BlockSpec((c_in, c_out), lambda i: (0, 0)),
        ],
        out_specs=pl.BlockSpec((blk, c_out), lambda i: (i, 0)),
        out_shape=jax.ShapeDtypeStruct((n, c_out), jnp.float32),
    )(ref_feat, w)


_NW = 32
_CH = 80


def _gather_sum(proj, idx_w, nq_pad):
    c = proj.shape[1]
    b_per_w = nq_pad // _NW
    n_chunks = b_per_w // _CH

    mesh = plsc.VectorSubcoreMesh(core_axis_name="c", subcore_axis_name="s")

    @functools.partial(
        pl.kernel,
        mesh=mesh,
        out_type=jax.ShapeDtypeStruct((nq_pad, c), jnp.float32),
        scratch_types=[
            pltpu.VMEM((3 * b_per_w,), jnp.int32),
            pltpu.VMEM((_CH, c), jnp.float32),
            pltpu.VMEM((_CH, c), jnp.float32),
            pltpu.VMEM((_CH, c), jnp.float32),
            pltpu.VMEM((_CH, c), jnp.float32),
            pltpu.VMEM((_CH, c), jnp.float32),
            pltpu.VMEM((_CH, c), jnp.float32),
            pltpu.SemaphoreType.DMA,
            pltpu.SemaphoreType.DMA,
            pltpu.SemaphoreType.DMA,
            pltpu.SemaphoreType.DMA,
        ],
    )
    def k3(proj_hbm, idx_hbm, out_hbm, idxv,
           r0a, r1a, r2a, r0b, r1b, r2b, sa, sb, oa, ob, ):
        wid = lax.axis_index("s") * 2 + lax.axis_index("c")
        base = wid * b_per_w
        pltpu.sync_copy(idx_hbm.at[wid], idxv)
        bufs = ((r0a, r1a, r2a, sa, oa), (r0b, r1b, r2b, sb, ob))

        def fire(ci):
            r0, r1, r2, sem, _ = bufs[ci % 2]
            off = ci * _CH
            return (
                pltpu.async_copy(proj_hbm.at[idxv.at[pl.ds(off, _CH)]],
                                 r0, sem),
                pltpu.async_copy(
                    proj_hbm.at[idxv.at[pl.ds(b_per_w + off, _CH)]], r1, sem),
                pltpu.async_copy(
                    proj_hbm.at[idxv.at[pl.ds(2 * b_per_w + off, _CH)]],
                    r2, sem),
            )

        gather_hs = {0: fire(0)}
        out_hs = {}
        for ci in range(n_chunks):
            r0, r1, r2, _, osem = bufs[ci % 2]
            if ci + 1 < n_chunks:
                if ci - 1 >= 0:
                    out_hs.pop(ci - 1).wait()
                gather_hs[ci + 1] = fire(ci + 1)
            for h in gather_hs.pop(ci):
                h.wait()

            def add_body(r, _):
                for j in range(c // 16):
                    sl = pl.ds(j * 16, 16)
                    r0[r, sl] = r0[r, sl] + r1[r, sl] + r2[r, sl]
                return 0

            lax.fori_loop(0, _CH, add_body, 0)
            out_hs[ci] = pltpu.async_copy(
                r0, out_hbm.at[pl.ds(base + ci * _CH, _CH)], osem)
        for ci in sorted(out_hs):
            out_hs[ci].wait()

    return k3(proj, idx_w)


def _final_body(s_ref, w_ref, g_ref, b_ref, o_ref):
    acc = jnp.dot(s_ref[...], w_ref[...], preferred_element_type=jnp.float32)
    o_ref[...] = jnp.maximum(acc + g_ref[...] + b_ref[...], 0.0)


def _final(skip, w, gsum, b):
    n, c_in = skip.shape
    c_out = w.shape[1]
    blk = next(bb for bb in (400, 200, 8) if n % bb == 0)
    return pl.pallas_call(
        _final_body,
        grid=(n // blk,),
        in_specs=[
            pl.BlockSpec((blk, c_in), lambda i: (i, 0)),
            pl.BlockSpec((c_in, c_out), lambda i: (0, 0)),
            pl.BlockSpec((blk, c_out), lambda i: (i, 0)),
            pl.BlockSpec((1, c_out), lambda i: (0, 0)),
        ],
        out_specs=pl.BlockSpec((blk, c_out), lambda i: (i, 0)),
        out_shape=jax.ShapeDtypeStruct((n, c_out), jnp.float32),
    )(skip, w, gsum, b.reshape(1, c_out))


def kernel(ref_bxyz, ref_feat, query_bxyz, query_skip_feat, W_ref, W_skip, b):
    n_ref = ref_feat.shape[0]
    n_query = query_bxyz.shape[0]

    qry = query_bxyz[:, 1:]
    rxyz = ref_bxyz[:, 1:]
    rt8 = jnp.pad(rxyz.T, ((0, 5), (0, _NR_PAD - n_ref)))
    r2 = jnp.sum(rxyz * rxyz, axis=1)
    r2 = jnp.pad(r2, (0, _NR_PAD - n_ref), constant_values=3e8)[None, :]

    proj = _proj(ref_feat, W_ref)

    n_slab = n_query // _SLABS
    outs = []
    for sl in range(_SLABS):
        qs = qry[sl * n_slab:(sl + 1) * n_slab]
        nq1 = ((n_slab + _BQ - 1) // _BQ) * _BQ
        q8 = jnp.pad(qs, ((0, nq1 - n_slab), (0, 5)))
        q2 = jnp.sum(qs * qs, axis=1, keepdims=True)
        q2 = jnp.pad(q2, ((0, nq1 - n_slab), (0, 7)), mode="edge")

        idx8 = _knn_top3(q8, rt8, q2, r2)
        idx3 = idx8[:n_slab, :K]

        nq2 = ((n_slab + _NW * _CH - 1) // (_NW * _CH)) * (_NW * _CH)
        b_per_w = nq2 // _NW
        idx_w = (jnp.pad(idx3, ((0, nq2 - n_slab), (0, 0)))
                 .reshape(_NW, b_per_w, K).transpose(0, 2, 1)
                 .reshape(_NW, K * b_per_w))
        gsum = _gather_sum(proj, idx_w, nq2)[:n_slab]

        skip = query_skip_feat[sl * n_slab:(sl + 1) * n_slab]
        outs.append(_final(skip, W_skip, gsum, b))
    return jnp.concatenate(outs, axis=0)


# speedup vs baseline: 1.1037x; 1.0306x over previous
import functools

import jax
import jax.numpy as jnp
from jax import lax
from jax.experimental import pallas as pl
from jax.experimental.pallas import tpu as pltpu
from jax.experimental.pallas import tpu_sc as plsc

K = 3
_SLABS = 2


_BQ = 512
_NR_PAD = 10240


def _knn_body(q_ref, rt_ref, q2_ref, r2_ref, o_ref):
    m = jnp.dot(q_ref[...], rt_ref[...], preferred_element_type=jnp.float32)
    s = (q2_ref[...][:, 0:1] + r2_ref[...]) - 2.0 * m
    idx = lax.broadcasted_iota(jnp.int32, s.shape, 1)
    cols = []
    for t in range(K):
        mn = jnp.min(s, axis=1, keepdims=True)
        i = jnp.min(jnp.where(s == mn, idx, _NR_PAD), axis=1, keepdims=True)
        cols.append(i)
        if t < K - 1:
            s = jnp.where(idx == i, jnp.float32(jnp.inf), s)
    o_ref[...] = jnp.concatenate(cols + [cols[-1]] * (8 - K), axis=1)


def _knn_top3(q8, rt8, q2, r2):
    nq = q8.shape[0]
    grid = nq // _BQ
    return pl.pallas_call(
        _knn_body,
        grid=(grid,),
        in_specs=[
            pl.BlockSpec((_BQ, 8), lambda i: (i, 0)),
            pl.BlockSpec((8, _NR_PAD), lambda i: (0, 0)),
            pl.BlockSpec((_BQ, 8), lambda i: (i, 0)),
            pl.BlockSpec((1, _NR_PAD), lambda i: (0, 0)),
        ],
        out_specs=pl.BlockSpec((_BQ, 8), lambda i: (i, 0)),
        out_shape=jax.ShapeDtypeStruct((nq, 8), jnp.int32),
    )(q8, rt8, q2, r2)


def _proj_body(x_ref, w_ref, o_ref):
    o_ref[...] = jnp.dot(x_ref[...], w_ref[...],
                         preferred_element_type=jnp.float32) * (1.0 / K)


def _proj(ref_feat, w):
    n, c_in = ref_feat.shape
    c_out = w.shape[1]
    blk = 1000
    return pl.pallas_call(
        _proj_body,
        grid=(n // blk,),
        in_specs=[
            pl.BlockSpec((blk, c_in), lambda i: (i, 0)),
            pl.BlockSpec((c_in, c_out), lambda i: (0, 0)),
        ],
        out_specs=pl.BlockSpec((blk, c_out), lambda i: (i, 0)),
        out_shape=jax.ShapeDtypeStruct((n, c_out), jnp.float32),
    )(ref_feat, w)


_NW = 32
_CH = 80


def _gather_sum(proj, idx_w, nq_pad):
    c = proj.shape[1]
    b_per_w = nq_pad // _NW
    n_chunks = b_per_w // _CH

    mesh = plsc.VectorSubcoreMesh(core_axis_name="c", subcore_axis_name="s")

    @functools.partial(
        pl.kernel,
        mesh=mesh,
        out_type=jax.ShapeDtypeStruct((nq_pad, c), jnp.float32),
        scratch_types=[
            pltpu.VMEM((3 * b_per_w,), jnp.int32),
            pltpu.VMEM((_CH, c), jnp.float32),
            pltpu.VMEM((_CH, c), jnp.float32),
            pltpu.VMEM((_CH, c), jnp.float32),
            pltpu.VMEM((_CH, c), jnp.float32),
            pltpu.VMEM((_CH, c), jnp.float32),
            pltpu.VMEM((_CH, c), jnp.float32),
            pltpu.SemaphoreType.DMA,
            pltpu.SemaphoreType.DMA,
            pltpu.SemaphoreType.DMA,
            pltpu.SemaphoreType.DMA,
        ],
    )
    def k3(proj_hbm, idx_hbm, out_hbm, idxv,
           r0a, r1a, r2a, r0b, r1b, r2b, sa, sb, oa, ob, ):
        wid = lax.axis_index("s") * 2 + lax.axis_index("c")
        base = wid * b_per_w
        pltpu.sync_copy(idx_hbm.at[wid], idxv)
        bufs = ((r0a, r1a, r2a, sa, oa), (r0b, r1b, r2b, sb, ob))

        def fire(ci):
            r0, r1, r2, sem, _ = bufs[ci % 2]
            off = ci * _CH
            return (
                pltpu.async_copy(proj_hbm.at[idxv.at[pl.ds(off, _CH)]],
                                 r0, sem),
                pltpu.async_copy(
                    proj_hbm.at[idxv.at[pl.ds(b_per_w + off, _CH)]], r1, sem),
                pltpu.async_copy(
                    proj_hbm.at[idxv.at[pl.ds(2 * b_per_w + off, _CH)]],
                    r2, sem),
            )

        gather_hs = {0: fire(0)}
        out_hs = {}
        for ci in range(n_chunks):
            r0, r1, r2, _, osem = bufs[ci % 2]
            if ci + 1 < n_chunks:
                if ci - 1 >= 0:
                    out_hs.pop(ci - 1).wait()
                gather_hs[ci + 1] = fire(ci + 1)
            for h in gather_hs.pop(ci):
                h.wait()

            def add_body(r, _):
                for j in range(c // 16):
                    sl = pl.ds(j * 16, 16)
                    r0[r, sl] = r0[r, sl] + r1[r, sl] + r2[r, sl]
                return 0

            lax.fori_loop(0, _CH, add_body, 0)
            out_hs[ci] = pltpu.async_copy(
                r0, out_hbm.at[pl.ds(base + ci * _CH, _CH)], osem)
        for ci in sorted(out_hs):
            out_hs[ci].wait()

    return k3(proj, idx_w)


def _final_body(s_ref, w_ref, g_ref, b_ref, o_ref):
    acc = jnp.dot(s_ref[...], w_ref[...], preferred_element_type=jnp.float32)
    o_ref[...] = jnp.maximum(acc + g_ref[...] + b_ref[...], 0.0)


def _final(skip, w, gsum, b):
    n, c_in = skip.shape
    c_out = w.shape[1]
    blk = next(bb for bb in (400, 200, 8) if n % bb == 0)
    return pl.pallas_call(
        _final_body,
        grid=(n // blk,),
        in_specs=[
            pl.BlockSpec((blk, c_in), lambda i: (i, 0)),
            pl.BlockSpec((c_in, c_out), lambda i: (0, 0)),
            pl.BlockSpec((blk, c_out), lambda i: (i, 0)),
            pl.BlockSpec((1, c_out), lambda i: (0, 0)),
        ],
        out_specs=pl.BlockSpec((blk, c_out), lambda i: (i, 0)),
        out_shape=jax.ShapeDtypeStruct((n, c_out), jnp.float32),
    )(skip, w, gsum, b.reshape(1, c_out))


def kernel(ref_bxyz, ref_feat, query_bxyz, query_skip_feat, W_ref, W_skip, b):
    n_ref = ref_feat.shape[0]
    n_query = query_bxyz.shape[0]

    qry = query_bxyz[:, 1:]
    rxyz = ref_bxyz[:, 1:]
    rt8 = jnp.pad(rxyz.T, ((0, 5), (0, _NR_PAD - n_ref)))
    r2 = jnp.sum(rxyz * rxyz, axis=1)
    r2 = jnp.pad(r2, (0, _NR_PAD - n_ref), constant_values=3e8)[None, :]

    proj = _proj(ref_feat, W_ref)

    n_slab = n_query // _SLABS
    outs = []
    for sl in range(_SLABS):
        qs = qry[sl * n_slab:(sl + 1) * n_slab]
        nq1 = ((n_slab + _BQ - 1) // _BQ) * _BQ
        q8 = jnp.pad(qs, ((0, nq1 - n_slab), (0, 5)))
        q2 = jnp.sum(qs * qs, axis=1, keepdims=True)
        q2 = jnp.pad(q2, ((0, nq1 - n_slab), (0, 7)), mode="edge")

        idx8 = _knn_top3(q8, rt8, q2, r2)
        idx3 = idx8[:n_slab, :K]

        nq2 = ((n_slab + _NW * _CH - 1) // (_NW * _CH)) * (_NW * _CH)
        b_per_w = nq2 // _NW
        idx_w = (jnp.pad(idx3, ((0, nq2 - n_slab), (0, 0)))
                 .reshape(_NW, b_per_w, K).transpose(0, 2, 1)
                 .reshape(_NW, K * b_per_w))
        gsum = _gather_sum(proj, idx_w, nq2)[:n_slab]

        skip = query_skip_feat[sl * n_slab:(sl + 1) * n_slab]
        outs.append(_final(skip, W_skip, gsum, b))
    return jnp.concatenate(outs, axis=0)
